# baseline (speedup 1.0000x reference)
import functools

import jax
import jax.numpy as jnp
from jax import lax
from jax.experimental import pallas as pl
from jax.experimental.pallas import tpu as pltpu
from jax.experimental.pallas import tpu_sc as plsc

T = 2048
D_MODEL = 1024
D_FF = 2048
E = 8
TOPK = 2

BLK = 256
NB = (T * TOPK) // BLK + E
P_CAP = NB * BLK

NC, NS = 2, 16
NW = NC * NS
_GCH = 48
_GNC = P_CAP // NW // _GCH


def _dispatch_metadata(gating_output):
    probs = jax.nn.softmax(gating_output.astype(jnp.float32), axis=-1)
    topk_w, topk_idx = jax.lax.top_k(probs, TOPK)
    topk_w = topk_w / jnp.sum(topk_w, axis=-1, keepdims=True)

    e_pair = topk_idx.reshape(-1)
    w_pair = topk_w.reshape(-1)
    t_pair = jnp.arange(T * TOPK, dtype=jnp.int32) // TOPK

    onehot = (e_pair[:, None] == jnp.arange(E, dtype=e_pair.dtype)[None, :])
    counts = jnp.sum(onehot.astype(jnp.int32), axis=0)
    rank = jnp.cumsum(onehot.astype(jnp.int32), axis=0) - 1
    rank_in_e = jnp.take_along_axis(rank, e_pair[:, None].astype(jnp.int32),
                                    axis=1)[:, 0]

    blocks_per_e = (counts + BLK - 1) // BLK
    ends_blocks = jnp.cumsum(blocks_per_e)
    starts = jnp.concatenate(
        [jnp.zeros((1,), jnp.int32), ends_blocks[:-1]]) * BLK
    slot = starts[e_pair] + rank_in_e

    src_token = jnp.zeros((P_CAP,), jnp.int32).at[slot].set(t_pair)
    w_slot = jnp.zeros((P_CAP,), jnp.float32).at[slot].set(w_pair)

    block_expert = jnp.clip(
        jnp.searchsorted(ends_blocks, jnp.arange(NB), side="right"),
        0, E - 1).astype(jnp.int32)
    n_used = ends_blocks[-1:].astype(jnp.int32)
    block_expert = jnp.where(jnp.arange(NB) < n_used[0], block_expert,
                             block_expert[n_used[0] - 1]).astype(jnp.int32)
    return src_token, w_slot, slot.astype(jnp.int32), block_expert, n_used


def _tc_stage_x(x):
    def body(x_ref, o_ref):
        o_ref[...] = x_ref[...]

    return pl.pallas_call(
        body,
        grid=(8,),
        in_specs=[pl.BlockSpec((T // 8, D_MODEL), lambda i: (i, 0))],
        out_specs=pl.BlockSpec((T // 8, D_MODEL), lambda i: (i, 0)),
        out_shape=jax.ShapeDtypeStruct((T, D_MODEL), jnp.float32),
    )(x)


def _sc_gather(x, src_token2d):
    b_per_w = P_CAP // NW

    @functools.partial(
        pl.kernel,
        mesh=plsc.VectorSubcoreMesh(core_axis_name="c", subcore_axis_name="s"),
        out_type=jax.ShapeDtypeStruct((P_CAP, D_MODEL), jnp.float32),
        scratch_types=[
            pltpu.VMEM((_GNC, _GCH), jnp.int32),
            pltpu.VMEM((_GCH, D_MODEL), jnp.float32),
            pltpu.VMEM((_GCH, D_MODEL), jnp.float32),
            pltpu.SemaphoreType.DMA,
            pltpu.SemaphoreType.DMA,
            pltpu.SemaphoreType.DMA,
            pltpu.SemaphoreType.DMA,
        ],
    )
    def gather_k(x_hbm, idx_hbm, out_hbm, idx_v, r0, r1, sg0, sg1, so0, so1):
        wid = lax.axis_index("s") * NC + lax.axis_index("c")
        base = wid * b_per_w
        pltpu.sync_copy(idx_hbm.at[pl.ds(wid * _GNC, _GNC)], idx_v)
        rows = (r0, r1)
        sg = (sg0, sg1)
        so = (so0, so1)
        out_cp = [None, None]
        for c in range(_GNC):
            buf = c % 2
            if out_cp[buf] is not None:
                out_cp[buf].wait()
            pltpu.async_copy(x_hbm.at[idx_v.at[c]], rows[buf], sg[buf]).wait()
            out_cp[buf] = pltpu.async_copy(
                rows[buf], out_hbm.at[pl.ds(base + c * _GCH, _GCH)], so[buf])
        for buf in range(2):
            if out_cp[buf] is not None:
                out_cp[buf].wait()

    return gather_k(x, src_token2d)


def _gemm_body(e_ref, nu_ref, x_ref, g_ref, u_ref, d_ref, w_ref, out_ref):
    b = pl.program_id(0)

    @pl.when(b < nu_ref[0])
    def _():
        xb = x_ref[...]
        g = lax.dot_general(xb, g_ref[0], (((1,), (1,)), ((), ())),
                            preferred_element_type=jnp.float32)
        u = lax.dot_general(xb, u_ref[0], (((1,), (1,)), ((), ())),
                            preferred_element_type=jnp.float32)
        h = g * jax.nn.sigmoid(g) * u
        y = lax.dot_general(h, d_ref[0], (((1,), (1,)), ((), ())),
                            preferred_element_type=jnp.float32)
        out_ref[...] = y * w_ref[0, 0, :][:, None]


def _gemm_specs():
    def bm(b, e, nu):
        return jnp.minimum(b, nu[0] - 1)

    return dict(
        in_specs=[
            pl.BlockSpec((BLK, D_MODEL), lambda b, e, nu: (bm(b, e, nu), 0)),
            pl.BlockSpec((1, D_FF, D_MODEL),
                         lambda b, e, nu: (e[bm(b, e, nu)], 0, 0)),
            pl.BlockSpec((1, D_FF, D_MODEL),
                         lambda b, e, nu: (e[bm(b, e, nu)], 0, 0)),
            pl.BlockSpec((1, D_MODEL, D_FF),
                         lambda b, e, nu: (e[bm(b, e, nu)], 0, 0)),
            pl.BlockSpec((1, 1, BLK), lambda b, e, nu: (bm(b, e, nu), 0, 0)),
        ],
        out_specs=pl.BlockSpec((BLK, D_MODEL), lambda b, e, nu: (bm(b, e, nu), 0)),
    )


def _tc_gemm(block_expert, n_used, x_sorted, gate_proj, up_proj, down_proj, w3):
    specs = _gemm_specs()
    grid_spec = pltpu.PrefetchScalarGridSpec(
        num_scalar_prefetch=2,
        grid=(NB,),
        in_specs=specs["in_specs"],
        out_specs=specs["out_specs"],
    )
    return pl.pallas_call(
        _gemm_body,
        grid_spec=grid_spec,
        out_shape=jax.ShapeDtypeStruct((P_CAP, D_MODEL), jnp.float32),
        compiler_params=pltpu.CompilerParams(
            dimension_semantics=("arbitrary",),
            vmem_limit_bytes=100 * 1024 * 1024),
    )(block_expert, n_used, x_sorted, gate_proj, up_proj, down_proj, w3)


_CCH = 16


def _sc_combine(y_sorted, slot_pairs):
    t_per_w = T // NW

    @functools.partial(
        pl.kernel,
        mesh=plsc.VectorSubcoreMesh(core_axis_name="c", subcore_axis_name="s"),
        out_type=jax.ShapeDtypeStruct((T, D_MODEL), jnp.float32),
        scratch_types=[
            pltpu.VMEM((2 * _CCH,), jnp.int32),
            pltpu.VMEM((2 * _CCH, D_MODEL), jnp.float32),
            pltpu.VMEM((_CCH, D_MODEL), jnp.float32),
            pltpu.SemaphoreType.DMA,
        ],
    )
    def combine_k(y_hbm, pos_hbm, out_hbm, idx_v, rows_v, out_v, sem):
        wid = lax.axis_index("s") * NC + lax.axis_index("c")
        base_t = wid * t_per_w
        for c in range(t_per_w // _CCH):
            tok0 = base_t + c * _CCH
            pltpu.sync_copy(pos_hbm.at[pl.ds(tok0 * TOPK, TOPK * _CCH)], idx_v)
            pltpu.async_copy(y_hbm.at[idx_v], rows_v, sem).wait()

            def body(j, carry):
                for i in range(_CCH):
                    out_v[i, pl.ds(j * 16, 16)] = (
                        rows_v[2 * i, pl.ds(j * 16, 16)]
                        + rows_v[2 * i + 1, pl.ds(j * 16, 16)])
                return carry

            lax.fori_loop(0, D_MODEL // 16, body, 0)
            pltpu.sync_copy(out_v, out_hbm.at[pl.ds(tok0, _CCH)])

    return combine_k(y_sorted, slot_pairs)


def kernel(x, gating_output, gate_proj, up_proj, down_proj):
    src_token, w_slot, slot, block_expert, n_used = _dispatch_metadata(
        gating_output)
    w3 = w_slot.reshape(NB, 1, BLK)
    src_token2d = src_token.reshape(NW * _GNC, _GCH)

    x_sorted = _sc_gather(_tc_stage_x(x), src_token2d)
    y_sorted = _tc_gemm(block_expert, n_used, x_sorted,
                        gate_proj, up_proj, down_proj, w3)
    out = _sc_combine(y_sorted, slot)
    return out


# speedup vs baseline: 1.1713x; 1.1713x over previous
import functools

import jax
import jax.numpy as jnp
from jax import lax
from jax.experimental import pallas as pl
from jax.experimental.pallas import tpu as pltpu
from jax.experimental.pallas import tpu_sc as plsc

T = 2048
D_MODEL = 1024
D_FF = 2048
E = 8
TOPK = 2

BLK = 256
NB = (T * TOPK) // BLK + E
P_CAP = NB * BLK

NC, NS = 2, 16
NW = NC * NS
_GCH = 48
_GNC = P_CAP // NW // _GCH


def _dispatch_metadata(gating_output):
    probs = jax.nn.softmax(gating_output.astype(jnp.float32), axis=-1)
    topk_w, topk_idx = jax.lax.top_k(probs, TOPK)
    topk_w = topk_w / jnp.sum(topk_w, axis=-1, keepdims=True)

    e_pair = topk_idx.reshape(-1)
    w_pair = topk_w.reshape(-1)
    t_pair = jnp.arange(T * TOPK, dtype=jnp.int32) // TOPK

    onehot = (e_pair[:, None] == jnp.arange(E, dtype=e_pair.dtype)[None, :])
    counts = jnp.sum(onehot.astype(jnp.int32), axis=0)
    rank = jnp.cumsum(onehot.astype(jnp.int32), axis=0) - 1
    rank_in_e = jnp.take_along_axis(rank, e_pair[:, None].astype(jnp.int32),
                                    axis=1)[:, 0]

    blocks_per_e = (counts + BLK - 1) // BLK
    ends_blocks = jnp.cumsum(blocks_per_e)
    starts = jnp.concatenate(
        [jnp.zeros((1,), jnp.int32), ends_blocks[:-1]]) * BLK
    slot = starts[e_pair] + rank_in_e

    pad_rows = (jnp.arange(P_CAP, dtype=jnp.int32) * 193) % T
    src_token = pad_rows.at[slot].set(t_pair)
    w_slot = jnp.zeros((P_CAP,), jnp.float32).at[slot].set(w_pair)

    block_expert = jnp.clip(
        jnp.searchsorted(ends_blocks, jnp.arange(NB), side="right"),
        0, E - 1).astype(jnp.int32)
    n_used = ends_blocks[-1:].astype(jnp.int32)
    block_expert = jnp.where(jnp.arange(NB) < n_used[0], block_expert,
                             block_expert[n_used[0] - 1]).astype(jnp.int32)
    return src_token, w_slot, slot.astype(jnp.int32), block_expert, n_used


def _sc_gather(x, src_token2d):
    b_per_w = P_CAP // NW

    @functools.partial(
        pl.kernel,
        mesh=plsc.VectorSubcoreMesh(core_axis_name="c", subcore_axis_name="s"),
        out_type=jax.ShapeDtypeStruct((P_CAP, D_MODEL), jnp.float32),
        scratch_types=[
            pltpu.VMEM((_GNC, _GCH), jnp.int32),
            pltpu.VMEM((_GCH, D_MODEL), jnp.float32),
            pltpu.VMEM((_GCH, D_MODEL), jnp.float32),
            pltpu.SemaphoreType.DMA,
            pltpu.SemaphoreType.DMA,
            pltpu.SemaphoreType.DMA,
            pltpu.SemaphoreType.DMA,
        ],
    )
    def gather_k(x_hbm, idx_hbm, out_hbm, idx_v, r0, r1, sg0, sg1, so0, so1):
        wid = lax.axis_index("s") * NC + lax.axis_index("c")
        base = wid * b_per_w
        pltpu.sync_copy(idx_hbm.at[pl.ds(wid * _GNC, _GNC)], idx_v)
        rows = (r0, r1)
        sg = (sg0, sg1)
        so = (so0, so1)
        out_cp = [None, None]
        for c in range(_GNC):
            buf = c % 2
            if out_cp[buf] is not None:
                out_cp[buf].wait()
            pltpu.async_copy(x_hbm.at[idx_v.at[c]], rows[buf], sg[buf]).wait()
            out_cp[buf] = pltpu.async_copy(
                rows[buf], out_hbm.at[pl.ds(base + c * _GCH, _GCH)], so[buf])
        for buf in range(2):
            if out_cp[buf] is not None:
                out_cp[buf].wait()

    return gather_k(x, src_token2d)


FBLK = 512
NF = D_FF // FBLK


def _gemm_body(e_ref, nu_ref, x_ref, g_ref, u_ref, d_ref, w_ref, out_ref,
               acc_ref):
    b = pl.program_id(0)
    f = pl.program_id(1)

    @pl.when(b < nu_ref[0])
    def _():
        xb = x_ref[...]
        g = lax.dot_general(xb, g_ref[0], (((1,), (1,)), ((), ())),
                            preferred_element_type=jnp.float32)
        u = lax.dot_general(xb, u_ref[0], (((1,), (1,)), ((), ())),
                            preferred_element_type=jnp.float32)
        h = g * jax.nn.sigmoid(g) * u
        y = lax.dot_general(h, d_ref[0], (((1,), (1,)), ((), ())),
                            preferred_element_type=jnp.float32)

        @pl.when(f == 0)
        def _():
            acc_ref[...] = y

        @pl.when(f > 0)
        def _():
            acc_ref[...] += y

        @pl.when(f == NF - 1)
        def _():
            out_ref[...] = acc_ref[...] * w_ref[0, 0, :][:, None]


def _gemm_specs():
    def bm(b, nu):
        return jnp.minimum(b, nu[0] - 1)

    def fs(b, f, nu):
        serp = jnp.where(b % 2 == 0, f, NF - 1 - f)
        last = jnp.where((nu[0] - 1) % 2 == 0, NF - 1, 0)
        return jnp.where(b < nu[0], serp, last)

    return dict(
        in_specs=[
            pl.BlockSpec((BLK, D_MODEL), lambda b, f, e, nu: (bm(b, nu), 0)),
            pl.BlockSpec((1, FBLK, D_MODEL),
                         lambda b, f, e, nu: (e[bm(b, nu)], fs(b, f, nu), 0)),
            pl.BlockSpec((1, FBLK, D_MODEL),
                         lambda b, f, e, nu: (e[bm(b, nu)], fs(b, f, nu), 0)),
            pl.BlockSpec((1, D_MODEL, FBLK),
                         lambda b, f, e, nu: (e[bm(b, nu)], 0, fs(b, f, nu))),
            pl.BlockSpec((1, 1, BLK), lambda b, f, e, nu: (bm(b, nu), 0, 0)),
        ],
        out_specs=pl.BlockSpec((BLK, D_MODEL),
                               lambda b, f, e, nu: (bm(b, nu), 0)),
    )


def _tc_gemm(block_expert, n_used, x_sorted, gate_proj, up_proj, down_proj, w3):
    specs = _gemm_specs()
    grid_spec = pltpu.PrefetchScalarGridSpec(
        num_scalar_prefetch=2,
        grid=(NB, NF),
        in_specs=specs["in_specs"],
        out_specs=specs["out_specs"],
        scratch_shapes=[pltpu.VMEM((BLK, D_MODEL), jnp.float32)],
    )
    return pl.pallas_call(
        _gemm_body,
        grid_spec=grid_spec,
        out_shape=jax.ShapeDtypeStruct((P_CAP, D_MODEL), jnp.float32),
        compiler_params=pltpu.CompilerParams(
            dimension_semantics=("arbitrary", "arbitrary"),
            vmem_limit_bytes=100 * 1024 * 1024),
    )(block_expert, n_used, x_sorted, gate_proj, up_proj, down_proj, w3)


_CCH = 16


def _sc_combine(y_sorted, slot_pairs):
    t_per_w = T // NW

    @functools.partial(
        pl.kernel,
        mesh=plsc.VectorSubcoreMesh(core_axis_name="c", subcore_axis_name="s"),
        out_type=jax.ShapeDtypeStruct((T, D_MODEL), jnp.float32),
        scratch_types=[
            pltpu.VMEM((2 * _CCH,), jnp.int32),
            pltpu.VMEM((2 * _CCH, D_MODEL), jnp.float32),
            pltpu.VMEM((_CCH, D_MODEL), jnp.float32),
            pltpu.SemaphoreType.DMA,
        ],
    )
    def combine_k(y_hbm, pos_hbm, out_hbm, idx_v, rows_v, out_v, sem):
        wid = lax.axis_index("s") * NC + lax.axis_index("c")
        base_t = wid * t_per_w
        for c in range(t_per_w // _CCH):
            tok0 = base_t + c * _CCH
            pltpu.sync_copy(pos_hbm.at[pl.ds(tok0 * TOPK, TOPK * _CCH)], idx_v)
            pltpu.async_copy(y_hbm.at[idx_v], rows_v, sem).wait()

            def body(j, carry):
                for i in range(_CCH):
                    out_v[i, pl.ds(j * 16, 16)] = (
                        rows_v[2 * i, pl.ds(j * 16, 16)]
                        + rows_v[2 * i + 1, pl.ds(j * 16, 16)])
                return carry

            lax.fori_loop(0, D_MODEL // 16, body, 0)
            pltpu.sync_copy(out_v, out_hbm.at[pl.ds(tok0, _CCH)])

    return combine_k(y_sorted, slot_pairs)


def kernel(x, gating_output, gate_proj, up_proj, down_proj):
    src_token, w_slot, slot, block_expert, n_used = _dispatch_metadata(
        gating_output)
    w3 = w_slot.reshape(NB, 1, BLK)
    src_token2d = src_token.reshape(NW * _GNC, _GCH)

    x_sorted = _sc_gather(x, src_token2d)
    y_sorted = _tc_gemm(block_expert, n_used, x_sorted,
                        gate_proj, up_proj, down_proj, w3)
    out = _sc_combine(y_sorted, slot)
    return out


# speedup vs baseline: 1.3090x; 1.1175x over previous
import functools

import jax
import jax.numpy as jnp
from jax import lax
from jax.experimental import pallas as pl
from jax.experimental.pallas import tpu as pltpu
from jax.experimental.pallas import tpu_sc as plsc

T = 2048
D_MODEL = 1024
D_FF = 2048
E = 8
TOPK = 2

BLK = 256
NB = (T * TOPK) // BLK + E
P_CAP = NB * BLK

NC, NS = 2, 16
NW = NC * NS
_GCH = 48
_GNC = P_CAP // NW // _GCH


def _dispatch_metadata(gating_output):
    probs = jax.nn.softmax(gating_output.astype(jnp.float32), axis=-1)
    topk_w, topk_idx = jax.lax.top_k(probs, TOPK)
    topk_w = topk_w / jnp.sum(topk_w, axis=-1, keepdims=True)

    e_pair = topk_idx.reshape(-1)
    w_pair = topk_w.reshape(-1)
    t_pair = jnp.arange(T * TOPK, dtype=jnp.int32) // TOPK

    onehot = (e_pair[:, None] == jnp.arange(E, dtype=e_pair.dtype)[None, :])
    counts = jnp.sum(onehot.astype(jnp.int32), axis=0)
    rank = jnp.cumsum(onehot.astype(jnp.int32), axis=0) - 1
    rank_in_e = jnp.take_along_axis(rank, e_pair[:, None].astype(jnp.int32),
                                    axis=1)[:, 0]

    blocks_per_e = (counts + BLK - 1) // BLK
    ends_blocks = jnp.cumsum(blocks_per_e)
    starts = jnp.concatenate(
        [jnp.zeros((1,), jnp.int32), ends_blocks[:-1]]) * BLK
    slot = starts[e_pair] + rank_in_e

    pad_rows = (jnp.arange(P_CAP, dtype=jnp.int32) * 193) % T
    src_token = pad_rows.at[slot].set(t_pair)
    w_slot = jnp.zeros((P_CAP,), jnp.float32).at[slot].set(w_pair)

    block_expert = jnp.clip(
        jnp.searchsorted(ends_blocks, jnp.arange(NB), side="right"),
        0, E - 1).astype(jnp.int32)
    n_used = ends_blocks[-1:].astype(jnp.int32)
    block_expert = jnp.where(jnp.arange(NB) < n_used[0], block_expert,
                             block_expert[n_used[0] - 1]).astype(jnp.int32)
    return src_token, w_slot, slot.astype(jnp.int32), block_expert, n_used


def _sc_gather(x, src_token2d):
    b_per_w = P_CAP // NW

    @functools.partial(
        pl.kernel,
        mesh=plsc.VectorSubcoreMesh(core_axis_name="c", subcore_axis_name="s"),
        out_type=jax.ShapeDtypeStruct((P_CAP, D_MODEL), jnp.float32),
        scratch_types=[
            pltpu.VMEM((_GNC, _GCH), jnp.int32),
            pltpu.VMEM((_GCH, D_MODEL), jnp.float32),
            pltpu.VMEM((_GCH, D_MODEL), jnp.float32),
            pltpu.SemaphoreType.DMA,
            pltpu.SemaphoreType.DMA,
            pltpu.SemaphoreType.DMA,
            pltpu.SemaphoreType.DMA,
        ],
    )
    def gather_k(x_hbm, idx_hbm, out_hbm, idx_v, r0, r1, sg0, sg1, so0, so1):
        wid = lax.axis_index("s") * NC + lax.axis_index("c")
        base = wid * b_per_w
        pltpu.sync_copy(idx_hbm.at[pl.ds(wid * _GNC, _GNC)], idx_v)
        rows = (r0, r1)
        sg = (sg0, sg1)
        so = (so0, so1)
        out_cp = [None, None]
        for c in range(_GNC):
            buf = c % 2
            if out_cp[buf] is not None:
                out_cp[buf].wait()
            pltpu.async_copy(x_hbm.at[idx_v.at[c]], rows[buf], sg[buf]).wait()
            out_cp[buf] = pltpu.async_copy(
                rows[buf], out_hbm.at[pl.ds(base + c * _GCH, _GCH)], so[buf])
        for buf in range(2):
            if out_cp[buf] is not None:
                out_cp[buf].wait()

    return gather_k(x, src_token2d)


FBLK = 1024
NF = D_FF // FBLK


def _gemm_body(e_ref, nu_ref, x_ref, g_ref, u_ref, d_ref, w_ref, out_ref,
               acc_ref):
    b = pl.program_id(0)
    f = pl.program_id(1)

    @pl.when(b < nu_ref[0])
    def _():
        xb = x_ref[...]
        g = lax.dot_general(xb, g_ref[0], (((1,), (1,)), ((), ())),
                            preferred_element_type=jnp.float32)
        u = lax.dot_general(xb, u_ref[0], (((1,), (1,)), ((), ())),
                            preferred_element_type=jnp.float32)
        h = g * jax.nn.sigmoid(g) * u
        y = lax.dot_general(h, d_ref[0], (((1,), (1,)), ((), ())),
                            preferred_element_type=jnp.float32)

        @pl.when(f == 0)
        def _():
            acc_ref[...] = y

        @pl.when(f > 0)
        def _():
            acc_ref[...] += y

        @pl.when(f == NF - 1)
        def _():
            out_ref[...] = acc_ref[...] * w_ref[0, 0, :][:, None]


def _gemm_specs():
    def bm(b, nu):
        return jnp.minimum(b, nu[0] - 1)

    def fs(b, f, nu):
        serp = jnp.where(b % 2 == 0, f, NF - 1 - f)
        last = jnp.where((nu[0] - 1) % 2 == 0, NF - 1, 0)
        return jnp.where(b < nu[0], serp, last)

    return dict(
        in_specs=[
            pl.BlockSpec((BLK, D_MODEL), lambda b, f, e, nu: (bm(b, nu), 0)),
            pl.BlockSpec((1, FBLK, D_MODEL),
                         lambda b, f, e, nu: (e[bm(b, nu)], fs(b, f, nu), 0)),
            pl.BlockSpec((1, FBLK, D_MODEL),
                         lambda b, f, e, nu: (e[bm(b, nu)], fs(b, f, nu), 0)),
            pl.BlockSpec((1, D_MODEL, FBLK),
                         lambda b, f, e, nu: (e[bm(b, nu)], 0, fs(b, f, nu))),
            pl.BlockSpec((1, 1, BLK), lambda b, f, e, nu: (bm(b, nu), 0, 0)),
        ],
        out_specs=pl.BlockSpec((BLK, D_MODEL),
                               lambda b, f, e, nu: (bm(b, nu), 0)),
    )


def _tc_gemm(block_expert, n_used, x_sorted, gate_proj, up_proj, down_proj, w3):
    specs = _gemm_specs()
    grid_spec = pltpu.PrefetchScalarGridSpec(
        num_scalar_prefetch=2,
        grid=(NB, NF),
        in_specs=specs["in_specs"],
        out_specs=specs["out_specs"],
        scratch_shapes=[pltpu.VMEM((BLK, D_MODEL), jnp.float32)],
    )
    return pl.pallas_call(
        _gemm_body,
        grid_spec=grid_spec,
        out_shape=jax.ShapeDtypeStruct((P_CAP, D_MODEL), jnp.float32),
        compiler_params=pltpu.CompilerParams(
            dimension_semantics=("arbitrary", "arbitrary"),
            vmem_limit_bytes=100 * 1024 * 1024),
    )(block_expert, n_used, x_sorted, gate_proj, up_proj, down_proj, w3)


_CCH = 16


def _sc_combine(y_sorted, slot_pairs):
    t_per_w = T // NW

    @functools.partial(
        pl.kernel,
        mesh=plsc.VectorSubcoreMesh(core_axis_name="c", subcore_axis_name="s"),
        out_type=jax.ShapeDtypeStruct((T, D_MODEL), jnp.float32),
        scratch_types=[
            pltpu.VMEM((2 * _CCH,), jnp.int32),
            pltpu.VMEM((2 * _CCH, D_MODEL), jnp.float32),
            pltpu.VMEM((_CCH, D_MODEL), jnp.float32),
            pltpu.SemaphoreType.DMA,
        ],
    )
    def combine_k(y_hbm, pos_hbm, out_hbm, idx_v, rows_v, out_v, sem):
        wid = lax.axis_index("s") * NC + lax.axis_index("c")
        base_t = wid * t_per_w
        for c in range(t_per_w // _CCH):
            tok0 = base_t + c * _CCH
            pltpu.sync_copy(pos_hbm.at[pl.ds(tok0 * TOPK, TOPK * _CCH)], idx_v)
            pltpu.async_copy(y_hbm.at[idx_v], rows_v, sem).wait()

            def body(j, carry):
                for i in range(_CCH):
                    out_v[i, pl.ds(j * 16, 16)] = (
                        rows_v[2 * i, pl.ds(j * 16, 16)]
                        + rows_v[2 * i + 1, pl.ds(j * 16, 16)])
                return carry

            lax.fori_loop(0, D_MODEL // 16, body, 0)
            pltpu.sync_copy(out_v, out_hbm.at[pl.ds(tok0, _CCH)])

    return combine_k(y_sorted, slot_pairs)


def kernel(x, gating_output, gate_proj, up_proj, down_proj):
    src_token, w_slot, slot, block_expert, n_used = _dispatch_metadata(
        gating_output)
    w3 = w_slot.reshape(NB, 1, BLK)
    src_token2d = src_token.reshape(NW * _GNC, _GCH)

    x_sorted = _sc_gather(x, src_token2d)
    y_sorted = _tc_gemm(block_expert, n_used, x_sorted,
                        gate_proj, up_proj, down_proj, w3)
    out = _sc_combine(y_sorted, slot)
    return out


# speedup vs baseline: 1.4940x; 1.1414x over previous
import functools

import jax
import jax.numpy as jnp
from jax import lax
from jax.experimental import pallas as pl
from jax.experimental.pallas import tpu as pltpu
from jax.experimental.pallas import tpu_sc as plsc

T = 2048
D_MODEL = 1024
D_FF = 2048
E = 8
TOPK = 2

BLK = 256
NB = (T * TOPK) // BLK + E
P_CAP = NB * BLK

NC, NS = 2, 16
NW = NC * NS
_GCH = 48
_GNC = P_CAP // NW // _GCH


def _dispatch_metadata(gating_output):
    probs = jax.nn.softmax(gating_output.astype(jnp.float32), axis=-1)
    topk_w, topk_idx = jax.lax.top_k(probs, TOPK)
    topk_w = topk_w / jnp.sum(topk_w, axis=-1, keepdims=True)

    e_pair = topk_idx.reshape(-1)
    w_pair = topk_w.reshape(-1)
    t_pair = jnp.arange(T * TOPK, dtype=jnp.int32) // TOPK

    onehot = (e_pair[:, None] == jnp.arange(E, dtype=e_pair.dtype)[None, :])
    counts = jnp.sum(onehot.astype(jnp.int32), axis=0)
    rank = jnp.cumsum(onehot.astype(jnp.int32), axis=0) - 1
    rank_in_e = jnp.take_along_axis(rank, e_pair[:, None].astype(jnp.int32),
                                    axis=1)[:, 0]

    blocks_per_e = (counts + BLK - 1) // BLK
    ends_blocks = jnp.cumsum(blocks_per_e)
    starts = jnp.concatenate(
        [jnp.zeros((1,), jnp.int32), ends_blocks[:-1]]) * BLK
    slot = starts[e_pair] + rank_in_e

    pad_rows = (jnp.arange(P_CAP, dtype=jnp.int32) * 193) % T
    src_token = pad_rows.at[slot].set(t_pair)
    w_slot = jnp.zeros((P_CAP,), jnp.float32).at[slot].set(w_pair)

    block_expert = jnp.clip(
        jnp.searchsorted(ends_blocks, jnp.arange(NB), side="right"),
        0, E - 1).astype(jnp.int32)
    n_used = ends_blocks[-1:].astype(jnp.int32)
    block_expert = jnp.where(jnp.arange(NB) < n_used[0], block_expert,
                             block_expert[n_used[0] - 1]).astype(jnp.int32)
    return src_token, w_slot, slot.astype(jnp.int32), block_expert, n_used


def _sc_gather(x, src_token2d):
    b_per_w = P_CAP // NW

    @functools.partial(
        pl.kernel,
        mesh=plsc.VectorSubcoreMesh(core_axis_name="c", subcore_axis_name="s"),
        out_type=jax.ShapeDtypeStruct((P_CAP, D_MODEL), jnp.float32),
        scratch_types=[
            pltpu.VMEM((_GNC, _GCH), jnp.int32),
            pltpu.VMEM((_GCH, D_MODEL), jnp.float32),
            pltpu.VMEM((_GCH, D_MODEL), jnp.float32),
            pltpu.SemaphoreType.DMA,
            pltpu.SemaphoreType.DMA,
            pltpu.SemaphoreType.DMA,
            pltpu.SemaphoreType.DMA,
        ],
    )
    def gather_k(x_hbm, idx_hbm, out_hbm, idx_v, r0, r1, sg0, sg1, so0, so1):
        wid = lax.axis_index("s") * NC + lax.axis_index("c")
        base = wid * b_per_w
        pltpu.sync_copy(idx_hbm.at[pl.ds(wid * _GNC, _GNC)], idx_v)
        rows = (r0, r1)
        sg = (sg0, sg1)
        so = (so0, so1)
        out_cp = [None, None]
        for c in range(_GNC):
            buf = c % 2
            if out_cp[buf] is not None:
                out_cp[buf].wait()
            pltpu.async_copy(x_hbm.at[idx_v.at[c]], rows[buf], sg[buf]).wait()
            out_cp[buf] = pltpu.async_copy(
                rows[buf], out_hbm.at[pl.ds(base + c * _GCH, _GCH)], so[buf])
        for buf in range(2):
            if out_cp[buf] is not None:
                out_cp[buf].wait()

    return gather_k(x, src_token2d)


FBLK = 2048
NF = D_FF // FBLK


def _gemm_body(e_ref, nu_ref, x_ref, g_ref, u_ref, d_ref, w_ref, out_ref,
               acc_ref):
    b = pl.program_id(0)
    f = pl.program_id(1)

    @pl.when(b < nu_ref[0])
    def _():
        xb = x_ref[...]
        g = lax.dot_general(xb, g_ref[0], (((1,), (1,)), ((), ())),
                            preferred_element_type=jnp.float32)
        u = lax.dot_general(xb, u_ref[0], (((1,), (1,)), ((), ())),
                            preferred_element_type=jnp.float32)
        h = g * jax.nn.sigmoid(g) * u
        y = lax.dot_general(h, d_ref[0], (((1,), (1,)), ((), ())),
                            preferred_element_type=jnp.float32)

        @pl.when(f == 0)
        def _():
            acc_ref[...] = y

        @pl.when(f > 0)
        def _():
            acc_ref[...] += y

        @pl.when(f == NF - 1)
        def _():
            out_ref[...] = acc_ref[...] * w_ref[0, 0, :][:, None]


def _gemm_specs():
    def bm(b, nu):
        return jnp.minimum(b, nu[0] - 1)

    def fs(b, f, nu):
        serp = jnp.where(b % 2 == 0, f, NF - 1 - f)
        last = jnp.where((nu[0] - 1) % 2 == 0, NF - 1, 0)
        return jnp.where(b < nu[0], serp, last)

    return dict(
        in_specs=[
            pl.BlockSpec((BLK, D_MODEL), lambda b, f, e, nu: (bm(b, nu), 0)),
            pl.BlockSpec((1, FBLK, D_MODEL),
                         lambda b, f, e, nu: (e[bm(b, nu)], fs(b, f, nu), 0)),
            pl.BlockSpec((1, FBLK, D_MODEL),
                         lambda b, f, e, nu: (e[bm(b, nu)], fs(b, f, nu), 0)),
            pl.BlockSpec((1, D_MODEL, FBLK),
                         lambda b, f, e, nu: (e[bm(b, nu)], 0, fs(b, f, nu))),
            pl.BlockSpec((1, 1, BLK), lambda b, f, e, nu: (bm(b, nu), 0, 0)),
        ],
        out_specs=pl.BlockSpec((BLK, D_MODEL),
                               lambda b, f, e, nu: (bm(b, nu), 0)),
    )


def _tc_gemm(block_expert, n_used, x_sorted, gate_proj, up_proj, down_proj, w3):
    specs = _gemm_specs()
    grid_spec = pltpu.PrefetchScalarGridSpec(
        num_scalar_prefetch=2,
        grid=(NB, NF),
        in_specs=specs["in_specs"],
        out_specs=specs["out_specs"],
        scratch_shapes=[pltpu.VMEM((BLK, D_MODEL), jnp.float32)],
    )
    return pl.pallas_call(
        _gemm_body,
        grid_spec=grid_spec,
        out_shape=jax.ShapeDtypeStruct((P_CAP, D_MODEL), jnp.float32),
        compiler_params=pltpu.CompilerParams(
            dimension_semantics=("arbitrary", "arbitrary"),
            vmem_limit_bytes=100 * 1024 * 1024),
    )(block_expert, n_used, x_sorted, gate_proj, up_proj, down_proj, w3)


_CCH = 16


def _sc_combine(y_sorted, slot_pairs):
    t_per_w = T // NW

    @functools.partial(
        pl.kernel,
        mesh=plsc.VectorSubcoreMesh(core_axis_name="c", subcore_axis_name="s"),
        out_type=jax.ShapeDtypeStruct((T, D_MODEL), jnp.float32),
        scratch_types=[
            pltpu.VMEM((2 * _CCH,), jnp.int32),
            pltpu.VMEM((2 * _CCH, D_MODEL), jnp.float32),
            pltpu.VMEM((_CCH, D_MODEL), jnp.float32),
            pltpu.SemaphoreType.DMA,
        ],
    )
    def combine_k(y_hbm, pos_hbm, out_hbm, idx_v, rows_v, out_v, sem):
        wid = lax.axis_index("s") * NC + lax.axis_index("c")
        base_t = wid * t_per_w
        for c in range(t_per_w // _CCH):
            tok0 = base_t + c * _CCH
            pltpu.sync_copy(pos_hbm.at[pl.ds(tok0 * TOPK, TOPK * _CCH)], idx_v)
            pltpu.async_copy(y_hbm.at[idx_v], rows_v, sem).wait()

            def body(j, carry):
                for i in range(_CCH):
                    out_v[i, pl.ds(j * 16, 16)] = (
                        rows_v[2 * i, pl.ds(j * 16, 16)]
                        + rows_v[2 * i + 1, pl.ds(j * 16, 16)])
                return carry

            lax.fori_loop(0, D_MODEL // 16, body, 0)
            pltpu.sync_copy(out_v, out_hbm.at[pl.ds(tok0, _CCH)])

    return combine_k(y_sorted, slot_pairs)


def kernel(x, gating_output, gate_proj, up_proj, down_proj):
    src_token, w_slot, slot, block_expert, n_used = _dispatch_metadata(
        gating_output)
    w3 = w_slot.reshape(NB, 1, BLK)
    src_token2d = src_token.reshape(NW * _GNC, _GCH)

    x_sorted = _sc_gather(x, src_token2d)
    y_sorted = _tc_gemm(block_expert, n_used, x_sorted,
                        gate_proj, up_proj, down_proj, w3)
    out = _sc_combine(y_sorted, slot)
    return out


# speedup vs baseline: 1.5357x; 1.0279x over previous
import functools

import jax
import jax.numpy as jnp
from jax import lax
from jax.experimental import pallas as pl
from jax.experimental.pallas import tpu as pltpu
from jax.experimental.pallas import tpu_sc as plsc

T = 2048
D_MODEL = 1024
D_FF = 2048
E = 8
TOPK = 2

BLK = 512
NB = (T * TOPK) // BLK + E
P_CAP = NB * BLK

NC, NS = 2, 16
NW = NC * NS
_GCH = 32
_GNC = P_CAP // NW // _GCH


def _dispatch_metadata(gating_output):
    probs = jax.nn.softmax(gating_output.astype(jnp.float32), axis=-1)
    topk_w, topk_idx = jax.lax.top_k(probs, TOPK)
    topk_w = topk_w / jnp.sum(topk_w, axis=-1, keepdims=True)

    e_pair = topk_idx.reshape(-1)
    w_pair = topk_w.reshape(-1)
    t_pair = jnp.arange(T * TOPK, dtype=jnp.int32) // TOPK

    onehot = (e_pair[:, None] == jnp.arange(E, dtype=e_pair.dtype)[None, :])
    counts = jnp.sum(onehot.astype(jnp.int32), axis=0)
    rank = jnp.cumsum(onehot.astype(jnp.int32), axis=0) - 1
    rank_in_e = jnp.take_along_axis(rank, e_pair[:, None].astype(jnp.int32),
                                    axis=1)[:, 0]

    blocks_per_e = (counts + BLK - 1) // BLK
    ends_blocks = jnp.cumsum(blocks_per_e)
    starts = jnp.concatenate(
        [jnp.zeros((1,), jnp.int32), ends_blocks[:-1]]) * BLK
    slot = starts[e_pair] + rank_in_e

    pad_rows = (jnp.arange(P_CAP, dtype=jnp.int32) * 193) % T
    src_token = pad_rows.at[slot].set(t_pair)
    w_slot = jnp.zeros((P_CAP,), jnp.float32).at[slot].set(w_pair)

    block_expert = jnp.clip(
        jnp.searchsorted(ends_blocks, jnp.arange(NB), side="right"),
        0, E - 1).astype(jnp.int32)
    n_used = ends_blocks[-1:].astype(jnp.int32)
    block_expert = jnp.where(jnp.arange(NB) < n_used[0], block_expert,
                             block_expert[n_used[0] - 1]).astype(jnp.int32)
    return src_token, w_slot, slot.astype(jnp.int32), block_expert, n_used


def _sc_gather(x, src_token2d):
    b_per_w = P_CAP // NW

    @functools.partial(
        pl.kernel,
        mesh=plsc.VectorSubcoreMesh(core_axis_name="c", subcore_axis_name="s"),
        out_type=jax.ShapeDtypeStruct((P_CAP, D_MODEL), jnp.float32),
        scratch_types=[
            pltpu.VMEM((_GNC, _GCH), jnp.int32),
            pltpu.VMEM((_GCH, D_MODEL), jnp.float32),
            pltpu.VMEM((_GCH, D_MODEL), jnp.float32),
            pltpu.SemaphoreType.DMA,
            pltpu.SemaphoreType.DMA,
            pltpu.SemaphoreType.DMA,
            pltpu.SemaphoreType.DMA,
        ],
    )
    def gather_k(x_hbm, idx_hbm, out_hbm, idx_v, r0, r1, sg0, sg1, so0, so1):
        wid = lax.axis_index("s") * NC + lax.axis_index("c")
        base = wid * b_per_w
        pltpu.sync_copy(idx_hbm.at[pl.ds(wid * _GNC, _GNC)], idx_v)
        rows = (r0, r1)
        sg = (sg0, sg1)
        so = (so0, so1)
        out_cp = [None, None]
        for c in range(_GNC):
            buf = c % 2
            if out_cp[buf] is not None:
                out_cp[buf].wait()
            pltpu.async_copy(x_hbm.at[idx_v.at[c]], rows[buf], sg[buf]).wait()
            out_cp[buf] = pltpu.async_copy(
                rows[buf], out_hbm.at[pl.ds(base + c * _GCH, _GCH)], so[buf])
        for buf in range(2):
            if out_cp[buf] is not None:
                out_cp[buf].wait()

    return gather_k(x, src_token2d)


FBLK = 2048
NF = D_FF // FBLK


def _gemm_body(e_ref, nu_ref, x_ref, g_ref, u_ref, d_ref, w_ref, out_ref,
               acc_ref):
    b = pl.program_id(0)
    f = pl.program_id(1)

    @pl.when(b < nu_ref[0])
    def _():
        xb = x_ref[...]
        g = lax.dot_general(xb, g_ref[0], (((1,), (1,)), ((), ())),
                            preferred_element_type=jnp.float32)
        u = lax.dot_general(xb, u_ref[0], (((1,), (1,)), ((), ())),
                            preferred_element_type=jnp.float32)
        h = g * jax.nn.sigmoid(g) * u
        y = lax.dot_general(h, d_ref[0], (((1,), (1,)), ((), ())),
                            preferred_element_type=jnp.float32)

        if NF == 1:
            out_ref[...] = y * w_ref[0, 0, :][:, None]
        else:
            @pl.when(f == 0)
            def _():
                acc_ref[...] = y

            @pl.when(f > 0)
            def _():
                acc_ref[...] += y

            @pl.when(f == NF - 1)
            def _():
                out_ref[...] = acc_ref[...] * w_ref[0, 0, :][:, None]


def _gemm_specs():
    def bm(b, nu):
        return jnp.minimum(b, nu[0] - 1)

    def fs(b, f, nu):
        serp = jnp.where(b % 2 == 0, f, NF - 1 - f)
        last = jnp.where((nu[0] - 1) % 2 == 0, NF - 1, 0)
        return jnp.where(b < nu[0], serp, last)

    return dict(
        in_specs=[
            pl.BlockSpec((BLK, D_MODEL), lambda b, f, e, nu: (bm(b, nu), 0)),
            pl.BlockSpec((1, FBLK, D_MODEL),
                         lambda b, f, e, nu: (e[bm(b, nu)], fs(b, f, nu), 0)),
            pl.BlockSpec((1, FBLK, D_MODEL),
                         lambda b, f, e, nu: (e[bm(b, nu)], fs(b, f, nu), 0)),
            pl.BlockSpec((1, D_MODEL, FBLK),
                         lambda b, f, e, nu: (e[bm(b, nu)], 0, fs(b, f, nu))),
            pl.BlockSpec((1, 1, BLK), lambda b, f, e, nu: (bm(b, nu), 0, 0)),
        ],
        out_specs=pl.BlockSpec((BLK, D_MODEL),
                               lambda b, f, e, nu: (bm(b, nu), 0)),
    )


def _tc_gemm(block_expert, n_used, x_sorted, gate_proj, up_proj, down_proj, w3):
    specs = _gemm_specs()
    grid_spec = pltpu.PrefetchScalarGridSpec(
        num_scalar_prefetch=2,
        grid=(NB, NF),
        in_specs=specs["in_specs"],
        out_specs=specs["out_specs"],
        scratch_shapes=[pltpu.VMEM((BLK, D_MODEL), jnp.float32)],
    )
    return pl.pallas_call(
        _gemm_body,
        grid_spec=grid_spec,
        out_shape=jax.ShapeDtypeStruct((P_CAP, D_MODEL), jnp.float32),
        compiler_params=pltpu.CompilerParams(
            dimension_semantics=("arbitrary", "arbitrary"),
            vmem_limit_bytes=100 * 1024 * 1024),
    )(block_expert, n_used, x_sorted, gate_proj, up_proj, down_proj, w3)


_CCH = 16


def _sc_combine(y_sorted, slot_pairs):
    t_per_w = T // NW

    @functools.partial(
        pl.kernel,
        mesh=plsc.VectorSubcoreMesh(core_axis_name="c", subcore_axis_name="s"),
        out_type=jax.ShapeDtypeStruct((T, D_MODEL), jnp.float32),
        scratch_types=[
            pltpu.VMEM((2 * _CCH,), jnp.int32),
            pltpu.VMEM((2 * _CCH, D_MODEL), jnp.float32),
            pltpu.VMEM((_CCH, D_MODEL), jnp.float32),
            pltpu.SemaphoreType.DMA,
        ],
    )
    def combine_k(y_hbm, pos_hbm, out_hbm, idx_v, rows_v, out_v, sem):
        wid = lax.axis_index("s") * NC + lax.axis_index("c")
        base_t = wid * t_per_w
        for c in range(t_per_w // _CCH):
            tok0 = base_t + c * _CCH
            pltpu.sync_copy(pos_hbm.at[pl.ds(tok0 * TOPK, TOPK * _CCH)], idx_v)
            pltpu.async_copy(y_hbm.at[idx_v], rows_v, sem).wait()

            def body(j, carry):
                for i in range(_CCH):
                    out_v[i, pl.ds(j * 16, 16)] = (
                        rows_v[2 * i, pl.ds(j * 16, 16)]
                        + rows_v[2 * i + 1, pl.ds(j * 16, 16)])
                return carry

            lax.fori_loop(0, D_MODEL // 16, body, 0)
            pltpu.sync_copy(out_v, out_hbm.at[pl.ds(tok0, _CCH)])

    return combine_k(y_sorted, slot_pairs)


def kernel(x, gating_output, gate_proj, up_proj, down_proj):
    src_token, w_slot, slot, block_expert, n_used = _dispatch_metadata(
        gating_output)
    w3 = w_slot.reshape(NB, 1, BLK)
    src_token2d = src_token.reshape(NW * _GNC, _GCH)

    x_sorted = _sc_gather(x, src_token2d)
    y_sorted = _tc_gemm(block_expert, n_used, x_sorted,
                        gate_proj, up_proj, down_proj, w3)
    out = _sc_combine(y_sorted, slot)
    return out


# speedup vs baseline: 1.5696x; 1.0220x over previous
import functools

import jax
import jax.numpy as jnp
from jax import lax
from jax.experimental import pallas as pl
from jax.experimental.pallas import tpu as pltpu
from jax.experimental.pallas import tpu_sc as plsc

T = 2048
D_MODEL = 1024
D_FF = 2048
E = 8
TOPK = 2

BLK = 512
NB = (T * TOPK) // BLK + E
P_CAP = NB * BLK

NC, NS = 2, 16
NW = NC * NS
_GCH = 32
_GNC = P_CAP // NW // _GCH


def _dispatch_metadata(gating_output):
    probs = jax.nn.softmax(gating_output.astype(jnp.float32), axis=-1)
    iota_e = jnp.arange(E, dtype=jnp.int32)[None, :]
    m1 = jnp.max(probs, axis=-1, keepdims=True)
    i1 = jnp.min(jnp.where(probs == m1, iota_e, E), axis=-1, keepdims=True)
    pm = jnp.where(iota_e == i1, -jnp.inf, probs)
    m2 = jnp.max(pm, axis=-1, keepdims=True)
    i2 = jnp.min(jnp.where(pm == m2, iota_e, E), axis=-1, keepdims=True)
    ssum = m1 + m2
    e_pair = jnp.concatenate([i1, i2], axis=1).reshape(-1)
    w_pair = jnp.concatenate([m1 / ssum, m2 / ssum], axis=1).reshape(-1)
    t_pair = jnp.arange(T * TOPK, dtype=jnp.int32) // TOPK

    oh = (e_pair[:, None] == iota_e).astype(jnp.float32)
    ohc = oh.reshape(32, (T * TOPK) // 32, E)
    n = ohc.shape[1]
    ltri = (jnp.arange(n)[:, None] > jnp.arange(n)[None, :]).astype(jnp.float32)
    c1 = jnp.einsum('ij,cje->cie', ltri, ohc)
    csum = ohc.sum(axis=1)
    l32 = (jnp.arange(32)[:, None] > jnp.arange(32)[None, :]).astype(
        jnp.float32)
    cbase = l32 @ csum
    rank_in_e = ((c1 + cbase[:, None, :]).reshape(T * TOPK, E) * oh).sum(-1)

    counts = csum.sum(axis=0)
    blocks_per_e = jnp.ceil(counts / BLK)
    ends_blocks = jnp.cumsum(blocks_per_e).astype(jnp.int32)
    starts = (jnp.concatenate([jnp.zeros((1,), jnp.float32),
                               jnp.cumsum(blocks_per_e)[:-1]]) * BLK)
    slot = (starts[e_pair] + rank_in_e).astype(jnp.int32)

    pad_rows = (jnp.arange(P_CAP, dtype=jnp.int32) * 193) % T
    src_token = pad_rows.at[slot].set(t_pair)
    w_slot = jnp.zeros((P_CAP,), jnp.float32).at[slot].set(w_pair)

    block_expert = jnp.clip(
        jnp.searchsorted(ends_blocks, jnp.arange(NB), side="right"),
        0, E - 1).astype(jnp.int32)
    n_used = ends_blocks[-1:].astype(jnp.int32)
    block_expert = jnp.where(jnp.arange(NB) < n_used[0], block_expert,
                             block_expert[n_used[0] - 1]).astype(jnp.int32)
    return src_token, w_slot, slot.astype(jnp.int32), block_expert, n_used


def _sc_gather(x, src_token2d):
    b_per_w = P_CAP // NW

    @functools.partial(
        pl.kernel,
        mesh=plsc.VectorSubcoreMesh(core_axis_name="c", subcore_axis_name="s"),
        out_type=jax.ShapeDtypeStruct((P_CAP, D_MODEL), jnp.float32),
        scratch_types=[
            pltpu.VMEM((_GNC, _GCH), jnp.int32),
            pltpu.VMEM((_GCH, D_MODEL), jnp.float32),
            pltpu.VMEM((_GCH, D_MODEL), jnp.float32),
            pltpu.SemaphoreType.DMA,
            pltpu.SemaphoreType.DMA,
            pltpu.SemaphoreType.DMA,
            pltpu.SemaphoreType.DMA,
        ],
    )
    def gather_k(x_hbm, idx_hbm, out_hbm, idx_v, r0, r1, sg0, sg1, so0, so1):
        wid = lax.axis_index("s") * NC + lax.axis_index("c")
        base = wid * b_per_w
        pltpu.sync_copy(idx_hbm.at[pl.ds(wid * _GNC, _GNC)], idx_v)
        rows = (r0, r1)
        sg = (sg0, sg1)
        so = (so0, so1)
        out_cp = [None, None]
        for c in range(_GNC):
            buf = c % 2
            if out_cp[buf] is not None:
                out_cp[buf].wait()
            pltpu.async_copy(x_hbm.at[idx_v.at[c]], rows[buf], sg[buf]).wait()
            out_cp[buf] = pltpu.async_copy(
                rows[buf], out_hbm.at[pl.ds(base + c * _GCH, _GCH)], so[buf])
        for buf in range(2):
            if out_cp[buf] is not None:
                out_cp[buf].wait()

    return gather_k(x, src_token2d)


FBLK = 2048
NF = D_FF // FBLK


def _gemm_body(e_ref, nu_ref, x_ref, g_ref, u_ref, d_ref, w_ref, out_ref,
               acc_ref):
    b = pl.program_id(0)
    f = pl.program_id(1)

    @pl.when(b < nu_ref[0])
    def _():
        xb = x_ref[...]
        g = lax.dot_general(xb, g_ref[0], (((1,), (1,)), ((), ())),
                            preferred_element_type=jnp.float32)
        u = lax.dot_general(xb, u_ref[0], (((1,), (1,)), ((), ())),
                            preferred_element_type=jnp.float32)
        h = g * jax.nn.sigmoid(g) * u
        y = lax.dot_general(h, d_ref[0], (((1,), (1,)), ((), ())),
                            preferred_element_type=jnp.float32)

        if NF == 1:
            out_ref[...] = y * w_ref[0, 0, :][:, None]
        else:
            @pl.when(f == 0)
            def _():
                acc_ref[...] = y

            @pl.when(f > 0)
            def _():
                acc_ref[...] += y

            @pl.when(f == NF - 1)
            def _():
                out_ref[...] = acc_ref[...] * w_ref[0, 0, :][:, None]


def _gemm_specs():
    def bm(b, nu):
        return jnp.minimum(b, nu[0] - 1)

    def fs(b, f, nu):
        serp = jnp.where(b % 2 == 0, f, NF - 1 - f)
        last = jnp.where((nu[0] - 1) % 2 == 0, NF - 1, 0)
        return jnp.where(b < nu[0], serp, last)

    return dict(
        in_specs=[
            pl.BlockSpec((BLK, D_MODEL), lambda b, f, e, nu: (bm(b, nu), 0)),
            pl.BlockSpec((1, FBLK, D_MODEL),
                         lambda b, f, e, nu: (e[bm(b, nu)], fs(b, f, nu), 0)),
            pl.BlockSpec((1, FBLK, D_MODEL),
                         lambda b, f, e, nu: (e[bm(b, nu)], fs(b, f, nu), 0)),
            pl.BlockSpec((1, D_MODEL, FBLK),
                         lambda b, f, e, nu: (e[bm(b, nu)], 0, fs(b, f, nu))),
            pl.BlockSpec((1, 1, BLK), lambda b, f, e, nu: (bm(b, nu), 0, 0)),
        ],
        out_specs=pl.BlockSpec((BLK, D_MODEL),
                               lambda b, f, e, nu: (bm(b, nu), 0)),
    )


def _tc_gemm(block_expert, n_used, x_sorted, gate_proj, up_proj, down_proj, w3):
    specs = _gemm_specs()
    grid_spec = pltpu.PrefetchScalarGridSpec(
        num_scalar_prefetch=2,
        grid=(NB, NF),
        in_specs=specs["in_specs"],
        out_specs=specs["out_specs"],
        scratch_shapes=[pltpu.VMEM((BLK, D_MODEL), jnp.float32)],
    )
    return pl.pallas_call(
        _gemm_body,
        grid_spec=grid_spec,
        out_shape=jax.ShapeDtypeStruct((P_CAP, D_MODEL), jnp.float32),
        compiler_params=pltpu.CompilerParams(
            dimension_semantics=("arbitrary", "arbitrary"),
            vmem_limit_bytes=100 * 1024 * 1024),
    )(block_expert, n_used, x_sorted, gate_proj, up_proj, down_proj, w3)


_CCH = 16


def _sc_combine(y_sorted, slot_pairs):
    t_per_w = T // NW

    @functools.partial(
        pl.kernel,
        mesh=plsc.VectorSubcoreMesh(core_axis_name="c", subcore_axis_name="s"),
        out_type=jax.ShapeDtypeStruct((T, D_MODEL), jnp.float32),
        scratch_types=[
            pltpu.VMEM((2 * _CCH,), jnp.int32),
            pltpu.VMEM((2 * _CCH, D_MODEL), jnp.float32),
            pltpu.VMEM((_CCH, D_MODEL), jnp.float32),
            pltpu.SemaphoreType.DMA,
        ],
    )
    def combine_k(y_hbm, pos_hbm, out_hbm, idx_v, rows_v, out_v, sem):
        wid = lax.axis_index("s") * NC + lax.axis_index("c")
        base_t = wid * t_per_w
        for c in range(t_per_w // _CCH):
            tok0 = base_t + c * _CCH
            pltpu.sync_copy(pos_hbm.at[pl.ds(tok0 * TOPK, TOPK * _CCH)], idx_v)
            pltpu.async_copy(y_hbm.at[idx_v], rows_v, sem).wait()

            def body(j, carry):
                for i in range(_CCH):
                    out_v[i, pl.ds(j * 16, 16)] = (
                        rows_v[2 * i, pl.ds(j * 16, 16)]
                        + rows_v[2 * i + 1, pl.ds(j * 16, 16)])
                return carry

            lax.fori_loop(0, D_MODEL // 16, body, 0)
            pltpu.sync_copy(out_v, out_hbm.at[pl.ds(tok0, _CCH)])

    return combine_k(y_sorted, slot_pairs)


def kernel(x, gating_output, gate_proj, up_proj, down_proj):
    src_token, w_slot, slot, block_expert, n_used = _dispatch_metadata(
        gating_output)
    w3 = w_slot.reshape(NB, 1, BLK)
    src_token2d = src_token.reshape(NW * _GNC, _GCH)

    x_sorted = _sc_gather(x, src_token2d)
    y_sorted = _tc_gemm(block_expert, n_used, x_sorted,
                        gate_proj, up_proj, down_proj, w3)
    out = _sc_combine(y_sorted, slot)
    return out


# speedup vs baseline: 1.6241x; 1.0347x over previous
import functools

import jax
import jax.numpy as jnp
from jax import lax
from jax.experimental import pallas as pl
from jax.experimental.pallas import tpu as pltpu
from jax.experimental.pallas import tpu_sc as plsc

T = 2048
D_MODEL = 1024
D_FF = 2048
E = 8
TOPK = 2

BLK = 512
NB = (T * TOPK) // BLK + E
P_CAP = NB * BLK

NC, NS = 2, 16
NW = NC * NS
_GCH = 32
_GNC = P_CAP // NW // _GCH


def _dispatch_metadata(gating_output):
    probs = jax.nn.softmax(gating_output.astype(jnp.float32), axis=-1)
    iota_e = jnp.arange(E, dtype=jnp.int32)[None, :]
    m1 = jnp.max(probs, axis=-1, keepdims=True)
    i1 = jnp.min(jnp.where(probs == m1, iota_e, E), axis=-1, keepdims=True)
    pm = jnp.where(iota_e == i1, -jnp.inf, probs)
    m2 = jnp.max(pm, axis=-1, keepdims=True)
    i2 = jnp.min(jnp.where(pm == m2, iota_e, E), axis=-1, keepdims=True)
    ssum = m1 + m2
    e_pair = jnp.concatenate([i1, i2], axis=1).reshape(-1)
    w_pair = jnp.concatenate([m1 / ssum, m2 / ssum], axis=1).reshape(-1)
    t_pair = jnp.arange(T * TOPK, dtype=jnp.int32) // TOPK

    oh = (e_pair[:, None] == iota_e).astype(jnp.float32)
    ohc = oh.reshape(32, (T * TOPK) // 32, E)
    n = ohc.shape[1]
    ltri = (jnp.arange(n)[:, None] > jnp.arange(n)[None, :]).astype(jnp.float32)
    c1 = jnp.einsum('ij,cje->cie', ltri, ohc)
    csum = ohc.sum(axis=1)
    l32 = (jnp.arange(32)[:, None] > jnp.arange(32)[None, :]).astype(
        jnp.float32)
    cbase = l32 @ csum
    rank_in_e = ((c1 + cbase[:, None, :]).reshape(T * TOPK, E) * oh).sum(-1)

    counts = csum.sum(axis=0)
    blocks_per_e = jnp.ceil(counts / BLK)
    ends_blocks = jnp.cumsum(blocks_per_e).astype(jnp.int32)
    starts = (jnp.concatenate([jnp.zeros((1,), jnp.float32),
                               jnp.cumsum(blocks_per_e)[:-1]]) * BLK)
    slot = (starts[e_pair] + rank_in_e).astype(jnp.int32)

    pad_rows = (jnp.arange(P_CAP, dtype=jnp.int32) * 193) % T
    src_token = pad_rows.at[slot].set(t_pair)
    w_slot = jnp.zeros((P_CAP,), jnp.float32).at[slot].set(w_pair)

    block_expert = jnp.clip(
        jnp.searchsorted(ends_blocks, jnp.arange(NB), side="right"),
        0, E - 1).astype(jnp.int32)
    n_used = ends_blocks[-1:].astype(jnp.int32)
    block_expert = jnp.where(jnp.arange(NB) < n_used[0], block_expert,
                             block_expert[n_used[0] - 1]).astype(jnp.int32)
    return src_token, w_slot, slot.astype(jnp.int32), block_expert, n_used


def _sc_gather(x, src_token2d):
    b_per_w = P_CAP // NW

    @functools.partial(
        pl.kernel,
        mesh=plsc.VectorSubcoreMesh(core_axis_name="c", subcore_axis_name="s"),
        out_type=jax.ShapeDtypeStruct((P_CAP, D_MODEL), jnp.float32),
        scratch_types=[
            pltpu.VMEM((_GNC, _GCH), jnp.int32),
            pltpu.VMEM((_GCH, D_MODEL), jnp.float32),
            pltpu.VMEM((_GCH, D_MODEL), jnp.float32),
            pltpu.SemaphoreType.DMA,
            pltpu.SemaphoreType.DMA,
            pltpu.SemaphoreType.DMA,
            pltpu.SemaphoreType.DMA,
        ],
    )
    def gather_k(x_hbm, idx_hbm, out_hbm, idx_v, r0, r1, sg0, sg1, so0, so1):
        wid = lax.axis_index("s") * NC + lax.axis_index("c")
        base = wid * b_per_w
        pltpu.sync_copy(idx_hbm.at[pl.ds(wid * _GNC, _GNC)], idx_v)
        rows = (r0, r1)
        sg = (sg0, sg1)
        so = (so0, so1)
        out_cp = [None, None]
        for c in range(_GNC):
            buf = c % 2
            if out_cp[buf] is not None:
                out_cp[buf].wait()
            pltpu.async_copy(x_hbm.at[idx_v.at[c]], rows[buf], sg[buf]).wait()
            out_cp[buf] = pltpu.async_copy(
                rows[buf], out_hbm.at[pl.ds(base + c * _GCH, _GCH)], so[buf])
        for buf in range(2):
            if out_cp[buf] is not None:
                out_cp[buf].wait()

    return gather_k(x, src_token2d)


FBLK = 2048
NF = D_FF // FBLK


def _gemm_body(e_ref, nu_ref, x_ref, g_ref, u_ref, d_ref, w_ref, out_ref,
               acc_ref):
    b = pl.program_id(0)
    f = pl.program_id(1)

    @pl.when(b < nu_ref[0])
    def _():
        xb = x_ref[...]
        g = lax.dot_general(xb, g_ref[0], (((1,), (1,)), ((), ())),
                            preferred_element_type=jnp.float32)
        u = lax.dot_general(xb, u_ref[0], (((1,), (1,)), ((), ())),
                            preferred_element_type=jnp.float32)
        h = g * jax.nn.sigmoid(g) * u
        y = lax.dot_general(h, d_ref[0], (((1,), (1,)), ((), ())),
                            preferred_element_type=jnp.float32)

        if NF == 1:
            out_ref[...] = y * w_ref[0, 0, :][:, None]
        else:
            @pl.when(f == 0)
            def _():
                acc_ref[...] = y

            @pl.when(f > 0)
            def _():
                acc_ref[...] += y

            @pl.when(f == NF - 1)
            def _():
                out_ref[...] = acc_ref[...] * w_ref[0, 0, :][:, None]


def _gemm_specs():
    def bm(b, nu):
        return jnp.minimum(b, nu[0] - 1)

    def fs(b, f, nu):
        serp = jnp.where(b % 2 == 0, f, NF - 1 - f)
        last = jnp.where((nu[0] - 1) % 2 == 0, NF - 1, 0)
        return jnp.where(b < nu[0], serp, last)

    return dict(
        in_specs=[
            pl.BlockSpec((BLK, D_MODEL), lambda b, f, e, nu: (bm(b, nu), 0)),
            pl.BlockSpec((1, FBLK, D_MODEL),
                         lambda b, f, e, nu: (e[bm(b, nu)], fs(b, f, nu), 0)),
            pl.BlockSpec((1, FBLK, D_MODEL),
                         lambda b, f, e, nu: (e[bm(b, nu)], fs(b, f, nu), 0)),
            pl.BlockSpec((1, D_MODEL, FBLK),
                         lambda b, f, e, nu: (e[bm(b, nu)], 0, fs(b, f, nu))),
            pl.BlockSpec((1, 1, BLK), lambda b, f, e, nu: (bm(b, nu), 0, 0)),
        ],
        out_specs=pl.BlockSpec((BLK, D_MODEL),
                               lambda b, f, e, nu: (bm(b, nu), 0)),
    )


def _tc_gemm(block_expert, n_used, x_sorted, gate_proj, up_proj, down_proj,
             w3):
    specs = _gemm_specs()
    grid_spec = pltpu.PrefetchScalarGridSpec(
        num_scalar_prefetch=2,
        grid=(NB, NF),
        in_specs=specs["in_specs"],
        out_specs=specs["out_specs"],
        scratch_shapes=[pltpu.VMEM((BLK, D_MODEL), jnp.float32)],
    )
    return pl.pallas_call(
        _gemm_body,
        grid_spec=grid_spec,
        out_shape=jax.ShapeDtypeStruct((P_CAP, D_MODEL), jnp.float32),
        compiler_params=pltpu.CompilerParams(
            dimension_semantics=("arbitrary", "arbitrary"),
            vmem_limit_bytes=100 * 1024 * 1024),
    )(block_expert, n_used, x_sorted, gate_proj, up_proj, down_proj, w3)


_CCH = 16
_CNC = (T // NW) // _CCH


def _sc_combine(y_sorted, slot_pairs):
    t_per_w = T // NW

    @functools.partial(
        pl.kernel,
        mesh=plsc.VectorSubcoreMesh(core_axis_name="c", subcore_axis_name="s"),
        out_type=jax.ShapeDtypeStruct((T, D_MODEL), jnp.float32),
        scratch_types=[
            pltpu.VMEM((TOPK * t_per_w,), jnp.int32),
            pltpu.VMEM((TOPK * _CCH, D_MODEL), jnp.float32),
            pltpu.VMEM((TOPK * _CCH, D_MODEL), jnp.float32),
            pltpu.VMEM((_CCH, D_MODEL), jnp.float32),
            pltpu.VMEM((_CCH, D_MODEL), jnp.float32),
            pltpu.SemaphoreType.DMA,
            pltpu.SemaphoreType.DMA,
            pltpu.SemaphoreType.DMA,
            pltpu.SemaphoreType.DMA,
        ],
    )
    def combine_k(y_hbm, pos_hbm, out_hbm, idx_v, r0, r1,
                  o0, o1, sg0, sg1, so0, so1):
        wid = lax.axis_index("s") * NC + lax.axis_index("c")
        base_t = wid * t_per_w
        pltpu.sync_copy(pos_hbm.at[pl.ds(base_t * TOPK, TOPK * t_per_w)],
                        idx_v)
        rows = (r0, r1)
        outs = (o0, o1)
        sg = (sg0, sg1)
        so = (so0, so1)
        gather_cp = [None, None]
        out_cp = [None, None]

        def start_gather(c):
            buf = c % 2
            gather_cp[buf] = pltpu.async_copy(
                y_hbm.at[idx_v.at[pl.ds(c * TOPK * _CCH, TOPK * _CCH)]],
                rows[buf], sg[buf])

        start_gather(0)
        for c in range(_CNC):
            buf = c % 2
            gather_cp[buf].wait()
            if c + 1 < _CNC:
                start_gather(c + 1)
            if out_cp[buf] is not None:
                out_cp[buf].wait()
            r_v = rows[buf]
            o_v = outs[buf]

            def body(j, carry):
                for i in range(_CCH):
                    o_v[i, pl.ds(j * 16, 16)] = (
                        r_v[2 * i, pl.ds(j * 16, 16)]
                        + r_v[2 * i + 1, pl.ds(j * 16, 16)])
                return carry

            lax.fori_loop(0, D_MODEL // 16, body, 0)
            out_cp[buf] = pltpu.async_copy(
                o_v, out_hbm.at[pl.ds(base_t + c * _CCH, _CCH)], so[buf])
        for buf in range(2):
            if out_cp[buf] is not None:
                out_cp[buf].wait()

    return combine_k(y_sorted, slot_pairs)


def kernel(x, gating_output, gate_proj, up_proj, down_proj):
    src_token, w_slot, slot, block_expert, n_used = _dispatch_metadata(
        gating_output)
    w3 = w_slot.reshape(NB, 1, BLK)
    src_token2d = src_token.reshape(NW * _GNC, _GCH)

    x_sorted = _sc_gather(x, src_token2d)
    y_sorted = _tc_gemm(block_expert, n_used, x_sorted,
                        gate_proj, up_proj, down_proj, w3)
    out = _sc_combine(y_sorted, slot)
    return out
